# baseline (speedup 1.0000x reference)
import functools

import jax
import jax.numpy as jnp
from jax import lax
from jax.experimental import pallas as pl
from jax.experimental.pallas import tpu as pltpu
from jax.experimental.pallas import tpu_sc as plsc

N_NODES = 10000
N_EDGES = 320000
D = 128
NUM_GRAPHS = 64
BN_EPS = 1e-5

NC = 2
NS = 16
NW = NC * NS
EPT = N_EDGES // NW
CH = 80
NCHUNK = EPT // CH
DEPTH = 3
NBODY = NCHUNK // DEPTH * DEPTH
ZROWS = N_NODES // 10


def _sc_segment_sum(z, e_flat, zeros_blk):
    mesh = plsc.VectorSubcoreMesh(core_axis_name="c", subcore_axis_name="s")

    @functools.partial(
        pl.kernel,
        mesh=mesh,
        out_type=jax.ShapeDtypeStruct((NC, N_NODES, D), jnp.float32),
        scratch_types=(
            [pltpu.VMEM((CH,), jnp.int32)] * (2 * DEPTH)
            + [pltpu.VMEM((CH, D), jnp.float32)] * DEPTH
            + [pltpu.VMEM_SHARED((N_NODES, D), jnp.float32)]
            + [pltpu.SemaphoreType.DMA] * (2 * DEPTH)
        ),
    )
    def k(z_hbm, e_hbm, zeros_hbm, out_hbm, *scr):
        sbuf = scr[0:DEPTH]
        dbuf = scr[DEPTH:2 * DEPTH]
        rows = scr[2 * DEPTH:3 * DEPTH]
        acc = scr[3 * DEPTH]
        semE = scr[3 * DEPTH + 1:4 * DEPTH + 1]
        semR = scr[4 * DEPTH + 1:5 * DEPTH + 1]
        c = lax.axis_index("c")
        s = lax.axis_index("s")
        w = s * NC + c
        base = w * EPT

        def edma(chunk, q):
            off = base + chunk * CH
            pltpu.async_copy(e_hbm.at[pl.ds(off, CH)], sbuf[q], semE[q])
            pltpu.async_copy(
                e_hbm.at[pl.ds(N_EDGES + off, CH)], dbuf[q], semE[q])

        def edma_wait(q):
            pltpu.make_async_copy(
                e_hbm.at[pl.ds(base, CH)], sbuf[q], semE[q]).wait()
            pltpu.make_async_copy(
                e_hbm.at[pl.ds(base, CH)], dbuf[q], semE[q]).wait()

        for q in range(DEPTH):
            edma(q, q)

        @pl.when(s < 10)
        def _():
            pltpu.sync_copy(zeros_hbm, rows[DEPTH - 1])
            for t in range(ZROWS // CH):
                pltpu.sync_copy(
                    rows[DEPTH - 1], acc.at[pl.ds(s * ZROWS + t * CH, CH)])
            rem = ZROWS % CH
            if rem:
                pltpu.sync_copy(
                    rows[DEPTH - 1].at[pl.ds(0, rem)],
                    acc.at[pl.ds(s * ZROWS + (ZROWS // CH) * CH, rem)])

        for q in range(DEPTH - 1):
            edma_wait(q)
            pltpu.async_copy(z_hbm.at[sbuf[q]], rows[q], semR[q])
        plsc.subcore_barrier()

        def body(i, carry):
            for q in range(DEPTH):
                j = DEPTH * i + q
                q2 = (q + DEPTH - 1) % DEPTH
                pltpu.make_async_copy(
                    z_hbm.at[sbuf[q]], rows[q], semR[q]).wait()
                edma_wait(q2)
                pltpu.async_copy(z_hbm.at[sbuf[q2]], rows[q2], semR[q2])
                pltpu.sync_copy(rows[q], acc.at[dbuf[q]], add=True)
                edma(jnp.minimum(j + DEPTH, NCHUNK - 1), q)
            return carry

        lax.fori_loop(0, NBODY // DEPTH, body, 0)
        for t in range(NCHUNK - NBODY):
            q = (NBODY + t) % DEPTH
            pltpu.make_async_copy(z_hbm.at[sbuf[q]], rows[q], semR[q]).wait()
            pltpu.sync_copy(rows[q], acc.at[dbuf[q]], add=True)
        for t in range(DEPTH - 1 - (NCHUNK - NBODY)):
            q = (NCHUNK + t) % DEPTH
            pltpu.make_async_copy(z_hbm.at[sbuf[q]], rows[q], semR[q]).wait()
        edma_wait((NBODY - 1) % DEPTH)
        plsc.subcore_barrier()

        @pl.when(s < 10)
        def _():
            pltpu.sync_copy(
                acc.at[pl.ds(s * ZROWS, ZROWS)],
                out_hbm.at[c].at[pl.ds(s * ZROWS, ZROWS)],
            )

    return k(z, e_flat, zeros_blk)


def _mlp_bn(z_in, agg_ref, w1_ref, b1_ref, w2_ref, b2_ref, gm_ref, bt_ref):
    h = z_in + agg_ref[0] + agg_ref[1]
    h = jnp.maximum(
        jnp.dot(h, w1_ref[...], preferred_element_type=jnp.float32) + b1_ref[...],
        0.0)
    h = jnp.dot(h, w2_ref[...], preferred_element_type=jnp.float32) + b2_ref[...]
    z = jnp.maximum(h, 0.0)
    mu = jnp.mean(z, axis=0, keepdims=True)
    var = jnp.mean(z * z, axis=0, keepdims=True) - mu * mu
    return (z - mu) * lax.rsqrt(var + BN_EPS) * gm_ref[...] + bt_ref[...]


def _mlp_bn_body(x_ref, agg_ref, w1_ref, b1_ref, w2_ref, b2_ref,
                 gm_ref, bt_ref, o_ref):
    o_ref[...] = _mlp_bn(x_ref[...], agg_ref, w1_ref, b1_ref, w2_ref, b2_ref,
                         gm_ref, bt_ref)


def _tc_layer(x, agg, w1, b1, w2, b2, gm, bt):
    return pl.pallas_call(
        _mlp_bn_body,
        out_shape=jax.ShapeDtypeStruct((N_NODES, D), jnp.float32),
    )(x, agg, w1, b1, w2, b2, gm, bt)


def _mlp_bn_pool_body(z1_ref, agg_ref, w1_ref, b1_ref, w2_ref, b2_ref,
                      gm_ref, bt_ref, batch_ref, zcat_ref, gcat_ref):
    z1 = z1_ref[...]
    z2 = _mlp_bn(z1, agg_ref, w1_ref, b1_ref, w2_ref, b2_ref, gm_ref, bt_ref)
    zcat_ref[:, :D] = z1
    zcat_ref[:, D:] = z2
    onehot_t = (lax.broadcasted_iota(jnp.int32, (NUM_GRAPHS, 1), 0)
                == batch_ref[...]).astype(jnp.float32)
    gcat_ref[:, :D] = jnp.dot(onehot_t, z1,
                              preferred_element_type=jnp.float32)
    gcat_ref[:, D:] = jnp.dot(onehot_t, z2,
                              preferred_element_type=jnp.float32)


def _tc_layer_pool(z1, agg, w1, b1, w2, b2, gm, bt, batch_row):
    return pl.pallas_call(
        _mlp_bn_pool_body,
        out_shape=(
            jax.ShapeDtypeStruct((N_NODES, 2 * D), jnp.float32),
            jax.ShapeDtypeStruct((NUM_GRAPHS, 2 * D), jnp.float32),
        ),
    )(z1, agg, w1, b1, w2, b2, gm, bt, batch_row)


def kernel(x, edge_index, batch, W1_0, b1_0, W2_0, b2_0, gamma_0, beta_0,
           W1_1, b1_1, W2_1, b2_1, gamma_1, beta_1):
    e_flat = edge_index.reshape(-1)
    zeros_blk = jnp.zeros((CH, D), jnp.float32)
    batch_row = batch.reshape(1, N_NODES)

    def row(v):
        return v.reshape(1, D)

    agg1 = _sc_segment_sum(x, e_flat, zeros_blk)
    z1 = _tc_layer(x, agg1, W1_0, row(b1_0), W2_0, row(b2_0),
                   row(gamma_0), row(beta_0))
    agg2 = _sc_segment_sum(z1, e_flat, zeros_blk)
    z_cat, g_cat = _tc_layer_pool(z1, agg2, W1_1, row(b1_1), W2_1, row(b2_1),
                                  row(gamma_1), row(beta_1), batch_row)
    return (z_cat, g_cat)


# speedup vs baseline: 1.0077x; 1.0077x over previous
import functools

import jax
import jax.numpy as jnp
from jax import lax
from jax.experimental import pallas as pl
from jax.experimental.pallas import tpu as pltpu
from jax.experimental.pallas import tpu_sc as plsc

N_NODES = 10000
N_EDGES = 320000
D = 128
NUM_GRAPHS = 64
BN_EPS = 1e-5

NC = 2
NS = 16
NW = NC * NS
EPT = N_EDGES // NW
CH = 80
NCHUNK = EPT // CH
DEPTH = 3
NBODY = NCHUNK // DEPTH * DEPTH
ZROWS = N_NODES // 10


def _sc_segment_sum(z, e_flat, zeros_blk):
    mesh = plsc.VectorSubcoreMesh(core_axis_name="c", subcore_axis_name="s")

    @functools.partial(
        pl.kernel,
        mesh=mesh,
        out_type=jax.ShapeDtypeStruct((NC, N_NODES, D), jnp.float32),
        scratch_types=(
            [pltpu.VMEM((CH,), jnp.int32)] * (2 * DEPTH)
            + [pltpu.VMEM((CH, D), jnp.float32)] * DEPTH
            + [pltpu.VMEM_SHARED((N_NODES, D), jnp.float32)]
            + [pltpu.SemaphoreType.DMA] * (2 * DEPTH)
        ),
    )
    def k(z_hbm, e_hbm, zeros_hbm, out_hbm, *scr):
        sbuf = scr[0:DEPTH]
        dbuf = scr[DEPTH:2 * DEPTH]
        rows = scr[2 * DEPTH:3 * DEPTH]
        acc = scr[3 * DEPTH]
        semE = scr[3 * DEPTH + 1:4 * DEPTH + 1]
        semR = scr[4 * DEPTH + 1:5 * DEPTH + 1]
        c = lax.axis_index("c")
        s = lax.axis_index("s")
        w = s * NC + c
        base = w * EPT

        def edma(chunk, q):
            off = base + chunk * CH
            pltpu.async_copy(e_hbm.at[pl.ds(off, CH)], sbuf[q], semE[q])
            pltpu.async_copy(
                e_hbm.at[pl.ds(N_EDGES + off, CH)], dbuf[q], semE[q])

        def edma_wait(q):
            pltpu.make_async_copy(
                e_hbm.at[pl.ds(base, CH)], sbuf[q], semE[q]).wait()
            pltpu.make_async_copy(
                e_hbm.at[pl.ds(base, CH)], dbuf[q], semE[q]).wait()

        for q in range(DEPTH):
            edma(q, q)

        @pl.when(s < 10)
        def _():
            pltpu.sync_copy(zeros_hbm, acc.at[pl.ds(s * ZROWS, ZROWS)])

        for q in range(DEPTH - 1):
            edma_wait(q)
            pltpu.async_copy(z_hbm.at[sbuf[q]], rows[q], semR[q])
        plsc.subcore_barrier()

        def body(i, carry):
            for q in range(DEPTH):
                j = DEPTH * i + q
                q2 = (q + DEPTH - 1) % DEPTH
                pltpu.make_async_copy(
                    z_hbm.at[sbuf[q]], rows[q], semR[q]).wait()
                edma_wait(q2)
                pltpu.async_copy(z_hbm.at[sbuf[q2]], rows[q2], semR[q2])
                pltpu.sync_copy(rows[q], acc.at[dbuf[q]], add=True)
                edma(jnp.minimum(j + DEPTH, NCHUNK - 1), q)
            return carry

        lax.fori_loop(0, NBODY // DEPTH, body, 0)
        for t in range(NCHUNK - NBODY):
            q = (NBODY + t) % DEPTH
            pltpu.make_async_copy(z_hbm.at[sbuf[q]], rows[q], semR[q]).wait()
            pltpu.sync_copy(rows[q], acc.at[dbuf[q]], add=True)
        for t in range(DEPTH - 1 - (NCHUNK - NBODY)):
            q = (NCHUNK + t) % DEPTH
            pltpu.make_async_copy(z_hbm.at[sbuf[q]], rows[q], semR[q]).wait()
        edma_wait((NBODY - 1) % DEPTH)
        plsc.subcore_barrier()

        @pl.when(s < 10)
        def _():
            pltpu.sync_copy(
                acc.at[pl.ds(s * ZROWS, ZROWS)],
                out_hbm.at[c].at[pl.ds(s * ZROWS, ZROWS)],
            )

    return k(z, e_flat, zeros_blk)


def _mlp_bn(z_in, agg_ref, w1_ref, b1_ref, w2_ref, b2_ref, gm_ref, bt_ref):
    h = z_in + agg_ref[0] + agg_ref[1]
    h = jnp.maximum(
        jnp.dot(h, w1_ref[...], preferred_element_type=jnp.float32) + b1_ref[...],
        0.0)
    h = jnp.dot(h, w2_ref[...], preferred_element_type=jnp.float32) + b2_ref[...]
    z = jnp.maximum(h, 0.0)
    mu = jnp.mean(z, axis=0, keepdims=True)
    var = jnp.mean(z * z, axis=0, keepdims=True) - mu * mu
    return (z - mu) * lax.rsqrt(var + BN_EPS) * gm_ref[...] + bt_ref[...]


def _mlp_bn_body(x_ref, agg_ref, w1_ref, b1_ref, w2_ref, b2_ref,
                 gm_ref, bt_ref, o_ref):
    o_ref[...] = _mlp_bn(x_ref[...], agg_ref, w1_ref, b1_ref, w2_ref, b2_ref,
                         gm_ref, bt_ref)


def _tc_layer(x, agg, w1, b1, w2, b2, gm, bt):
    return pl.pallas_call(
        _mlp_bn_body,
        out_shape=jax.ShapeDtypeStruct((N_NODES, D), jnp.float32),
    )(x, agg, w1, b1, w2, b2, gm, bt)


def _mlp_bn_pool_body(z1_ref, agg_ref, w1_ref, b1_ref, w2_ref, b2_ref,
                      gm_ref, bt_ref, batch_ref, zcat_ref, gcat_ref):
    z1 = z1_ref[...]
    z2 = _mlp_bn(z1, agg_ref, w1_ref, b1_ref, w2_ref, b2_ref, gm_ref, bt_ref)
    zcat_ref[:, :D] = z1
    zcat_ref[:, D:] = z2
    onehot_t = (lax.broadcasted_iota(jnp.int32, (NUM_GRAPHS, 1), 0)
                == batch_ref[...]).astype(jnp.float32)
    gcat_ref[:, :D] = jnp.dot(onehot_t, z1,
                              preferred_element_type=jnp.float32)
    gcat_ref[:, D:] = jnp.dot(onehot_t, z2,
                              preferred_element_type=jnp.float32)


def _tc_layer_pool(z1, agg, w1, b1, w2, b2, gm, bt, batch_row):
    return pl.pallas_call(
        _mlp_bn_pool_body,
        out_shape=(
            jax.ShapeDtypeStruct((N_NODES, 2 * D), jnp.float32),
            jax.ShapeDtypeStruct((NUM_GRAPHS, 2 * D), jnp.float32),
        ),
    )(z1, agg, w1, b1, w2, b2, gm, bt, batch_row)


def kernel(x, edge_index, batch, W1_0, b1_0, W2_0, b2_0, gamma_0, beta_0,
           W1_1, b1_1, W2_1, b2_1, gamma_1, beta_1):
    e_flat = edge_index.reshape(-1)
    zeros_blk = jnp.zeros((ZROWS, D), jnp.float32)
    batch_row = batch.reshape(1, N_NODES)

    def row(v):
        return v.reshape(1, D)

    agg1 = _sc_segment_sum(x, e_flat, zeros_blk)
    z1 = _tc_layer(x, agg1, W1_0, row(b1_0), W2_0, row(b2_0),
                   row(gamma_0), row(beta_0))
    agg2 = _sc_segment_sum(z1, e_flat, zeros_blk)
    z_cat, g_cat = _tc_layer_pool(z1, agg2, W1_1, row(b1_1), W2_1, row(b2_1),
                                  row(gamma_1), row(beta_1), batch_row)
    return (z_cat, g_cat)
